# Initial kernel scaffold; baseline (speedup 1.0000x reference)
import jax
import jax.numpy as jnp
from jax.experimental import pallas as pl


def kernel(h_species, h_reactions, rs_feat, sr_feat, W1_rs, b1_rs, W2_rs, b2_rs, g_s, be_s, W1_sr, b1_sr, W2_sr, b2_sr, g_r, be_r, rs_index, sr_index):
    raise NotImplementedError("write your pallas kernel here")


# speedup vs baseline: 1.8836x; 1.8836x over previous
import functools

import jax
import jax.numpy as jnp
from jax import lax
from jax.experimental import pallas as pl
from jax.experimental.pallas import tpu as pltpu
from jax.experimental.pallas import tpu_sc as plsc

_NS = 10000
_NR = 10000
_E = 160000
_D = 256
_EF = 16

_NC = 2
_NSUB = 16
_NW = _NC * _NSUB

_GCHUNK = 128
_SCHUNK = 80
_HALF = _D // _NC

@functools.cache
def _mesh():
    return plsc.VectorSubcoreMesh(core_axis_name="c", subcore_axis_name="s")


def _gather_body(table_hbm, idx_hbm, out_hbm, idx_v, rows_v, sem):
    c = lax.axis_index("c")
    s = lax.axis_index("s")
    wid = s * _NC + c
    nchunks = _E // _GCHUNK
    my_n = (nchunks - wid + _NW - 1) // _NW

    def body(i, carry):
        ch = wid + i * _NW
        base = pl.multiple_of(ch * _GCHUNK, 8)
        pltpu.sync_copy(idx_hbm.at[pl.ds(base, _GCHUNK)], idx_v)
        pltpu.async_copy(table_hbm.at[idx_v], rows_v, sem).wait()
        pltpu.sync_copy(rows_v, out_hbm.at[pl.ds(base, _GCHUNK)])
        return carry

    lax.fori_loop(0, my_n, body, 0)


def _sc_gather(table, idx):
    return pl.kernel(
        _gather_body,
        out_type=jax.ShapeDtypeStruct((_E, _D), jnp.float32),
        mesh=_mesh(),
        scratch_types=[
            pltpu.VMEM((_GCHUNK,), jnp.int32),
            pltpu.VMEM((_GCHUNK, _D), jnp.float32),
            pltpu.SemaphoreType.DMA,
        ],
        name="sc_gather_rows",
    )(table, idx)


def _scatter_body(msgs_hbm, idx_hbm, base_hbm, out_hbm,
                  idx_v, msg_v, acc_sh):
    c = lax.axis_index("c")
    s = lax.axis_index("s")
    col0 = pl.multiple_of(c * _HALF, _HALF)
    n_rows = base_hbm.shape[0]
    nrow_chunks = n_rows // _SCHUNK
    my_rows = (nrow_chunks - s + _NSUB - 1) // _NSUB

    def init_body(i, carry):
        r0 = pl.multiple_of((s + i * _NSUB) * _SCHUNK, 8)
        pltpu.sync_copy(base_hbm.at[pl.ds(r0, _SCHUNK), pl.ds(col0, _HALF)],
                        msg_v)
        pltpu.sync_copy(msg_v, acc_sh.at[pl.ds(r0, _SCHUNK)])
        return carry

    lax.fori_loop(0, my_rows, init_body, 0)
    plsc.subcore_barrier()

    nchunks = _E // _SCHUNK
    my_n = nchunks // _NSUB
    e0 = s * (_E // _NSUB)

    def body(i, carry):
        base = pl.multiple_of(e0 + i * _SCHUNK, 8)
        pltpu.sync_copy(idx_hbm.at[pl.ds(base, _SCHUNK)], idx_v)
        pltpu.sync_copy(msgs_hbm.at[pl.ds(base, _SCHUNK), pl.ds(col0, _HALF)],
                        msg_v)
        pltpu.sync_copy(msg_v, acc_sh.at[idx_v], add=True)
        return carry

    lax.fori_loop(0, my_n, body, 0)
    plsc.subcore_barrier()

    def out_body(i, carry):
        r0 = pl.multiple_of((s + i * _NSUB) * _SCHUNK, 8)
        pltpu.sync_copy(acc_sh.at[pl.ds(r0, _SCHUNK)], msg_v)
        pltpu.sync_copy(msg_v,
                        out_hbm.at[pl.ds(r0, _SCHUNK), pl.ds(col0, _HALF)])
        return carry

    lax.fori_loop(0, my_rows, out_body, 0)


def _sc_scatter_add(msgs, idx, base):
    n = base.shape[0]
    return pl.kernel(
        _scatter_body,
        out_type=jax.ShapeDtypeStruct((n, _D), jnp.float32),
        mesh=_mesh(),
        scratch_types=[
            pltpu.VMEM((_SCHUNK,), jnp.int32),
            pltpu.VMEM((_SCHUNK, _HALF), jnp.float32),
            pltpu.VMEM_SHARED((n, _HALF), jnp.float32),
        ],
        name="sc_scatter_add",
    )(msgs, idx, base)


def _layernorm(x, g, b, eps=1e-5):
    mu = jnp.mean(x, axis=-1, keepdims=True)
    var = jnp.mean((x - mu) ** 2, axis=-1, keepdims=True)
    return (x - mu) / jnp.sqrt(var + eps) * g + b


def _mlp_block(g_ref, f_ref, w1h_ref, w1f_ref, b1_ref, w2_ref, b2_ref,
               gam_ref, bet_ref, o_ref, *, fuse_ln):
    x = g_ref[...]
    if fuse_ln:
        x = _layernorm(x, gam_ref[...], bet_ref[...])
    a = (jnp.dot(x, w1h_ref[...], preferred_element_type=jnp.float32)
         + jnp.dot(f_ref[...], w1f_ref[...], preferred_element_type=jnp.float32)
         + b1_ref[...])
    h = a * jax.nn.sigmoid(a)
    o_ref[...] = (jnp.dot(h, w2_ref[...], preferred_element_type=jnp.float32)
                  + b2_ref[...])


def _tc_mlp(g, feat, w1h, w1f, b1, w2, b2, gam, bet, fuse_ln, block_e=1280):
    grid = _E // block_e
    row2 = lambda i: (0, 0)
    return pl.pallas_call(
        functools.partial(_mlp_block, fuse_ln=fuse_ln),
        grid=(grid,),
        in_specs=[
            pl.BlockSpec((block_e, _D), lambda i: (i, 0)),
            pl.BlockSpec((block_e, _EF), lambda i: (i, 0)),
            pl.BlockSpec((_D, _D), row2),
            pl.BlockSpec((_EF, _D), row2),
            pl.BlockSpec((1, _D), row2),
            pl.BlockSpec((_D, _D), row2),
            pl.BlockSpec((1, _D), row2),
            pl.BlockSpec((1, _D), row2),
            pl.BlockSpec((1, _D), row2),
        ],
        out_specs=pl.BlockSpec((block_e, _D), lambda i: (i, 0)),
        out_shape=jax.ShapeDtypeStruct((_E, _D), jnp.float32),
        name="tc_edge_mlp",
    )(g, feat, w1h, w1f, b1, w2, b2, gam, bet)


def _ln_block(x_ref, gam_ref, bet_ref, o_ref):
    o_ref[...] = _layernorm(x_ref[...], gam_ref[...], bet_ref[...])


def _tc_layernorm(x, gam, bet, block_n=2000):
    n = x.shape[0]
    grid = n // block_n
    return pl.pallas_call(
        _ln_block,
        grid=(grid,),
        in_specs=[
            pl.BlockSpec((block_n, _D), lambda i: (i, 0)),
            pl.BlockSpec((1, _D), lambda i: (0, 0)),
            pl.BlockSpec((1, _D), lambda i: (0, 0)),
        ],
        out_specs=pl.BlockSpec((block_n, _D), lambda i: (i, 0)),
        out_shape=jax.ShapeDtypeStruct((n, _D), jnp.float32),
        name="tc_layernorm",
    )(x, gam, bet)


def kernel(h_species, h_reactions, rs_feat, sr_feat,
           W1_rs, b1_rs, W2_rs, b2_rs, g_s, be_s,
           W1_sr, b1_sr, W2_sr, b2_sr, g_r, be_r,
           rs_index, sr_index):
    w1h_rs, w1f_rs = W1_rs[:_D], W1_rs[_D:]
    w1h_sr, w1f_sr = W1_sr[:_D], W1_sr[_D:]
    b1_rs2, b2_rs2 = b1_rs[None, :], b2_rs[None, :]
    b1_sr2, b2_sr2 = b1_sr[None, :], b2_sr[None, :]
    g_s2, be_s2 = g_s[None, :], be_s[None, :]
    g_r2, be_r2 = g_r[None, :], be_r[None, :]

    g1 = _sc_gather(h_reactions, rs_index[0])
    msgs1 = _tc_mlp(g1, rs_feat, w1h_rs, w1f_rs, b1_rs2, W2_rs, b2_rs2,
                    g_s2, be_s2, fuse_ln=False)
    s_raw = _sc_scatter_add(msgs1, rs_index[1], h_species)

    g2 = _sc_gather(s_raw, sr_index[0])
    msgs2 = _tc_mlp(g2, sr_feat, w1h_sr, w1f_sr, b1_sr2, W2_sr, b2_sr2,
                    g_s2, be_s2, fuse_ln=True)
    r_raw = _sc_scatter_add(msgs2, sr_index[1], h_reactions)

    h_species_out = _tc_layernorm(s_raw, g_s2, be_s2)
    h_reactions_out = _tc_layernorm(r_raw, g_r2, be_r2)
    return (h_species_out, h_reactions_out)


# speedup vs baseline: 2.4749x; 1.3140x over previous
import functools

import jax
import jax.numpy as jnp
from jax import lax
from jax.experimental import pallas as pl
from jax.experimental.pallas import tpu as pltpu
from jax.experimental.pallas import tpu_sc as plsc

_NS = 10000
_NR = 10000
_E = 160000
_D = 256
_EF = 16

_NC = 2
_NSUB = 16
_NW = _NC * _NSUB

_GC = 40
_GN = _E // _NW // _GC
_SC = 80
_SN = _E // _NSUB // _SC
_HALF = _D // _NC

@functools.cache
def _mesh():
    return plsc.VectorSubcoreMesh(core_axis_name="c", subcore_axis_name="s")


def _gather_body(table_hbm, idx_hbm, out_hbm, idx_v, buf, g0, g1, w0, w1):
    c = lax.axis_index("c")
    s = lax.axis_index("s")
    wid = s * _NC + c
    e0 = wid * (_GN * _GC)
    pltpu.sync_copy(idx_hbm.at[wid], idx_v)
    gsem = (g0, g1)
    wsem = (w0, w1)

    def fire_gather(ch, b):
        pltpu.async_copy(table_hbm.at[idx_v.at[ch]], buf.at[b], gsem[b])

    def drain_gather(b):
        pltpu.make_async_copy(table_hbm.at[idx_v.at[0]], buf.at[b],
                              gsem[b]).wait()

    def fire_wb(ch, b):
        base = pl.multiple_of(e0 + ch * _GC, 8)
        pltpu.async_copy(buf.at[b], out_hbm.at[pl.ds(base, _GC)], wsem[b])

    def drain_wb(b):
        pltpu.make_async_copy(buf.at[b], out_hbm.at[pl.ds(0, _GC)],
                              wsem[b]).wait()

    fire_gather(0, 0)
    fire_gather(1, 1)

    @pl.loop(2, _GN - 1, step=2)
    def _pipe(ch0):
        for b in (0, 1):
            ch = ch0 + b
            drain_gather(b)
            fire_wb(ch - 2, b)
            drain_wb(b)
            fire_gather(ch, b)

    drain_gather(0)
    fire_wb(_GN - 3, 0)
    drain_wb(0)
    fire_gather(_GN - 1, 0)
    drain_gather(1)
    fire_wb(_GN - 2, 1)
    drain_gather(0)
    fire_wb(_GN - 1, 0)
    drain_wb(1)
    drain_wb(0)


def _sc_gather(table, idx3):
    return pl.kernel(
        _gather_body,
        out_type=jax.ShapeDtypeStruct((_E, _D), jnp.float32),
        mesh=_mesh(),
        scratch_types=[
            pltpu.VMEM((_GN, _GC), jnp.int32),
            pltpu.VMEM((2, _GC, _D), jnp.float32),
            pltpu.SemaphoreType.DMA,
            pltpu.SemaphoreType.DMA,
            pltpu.SemaphoreType.DMA,
            pltpu.SemaphoreType.DMA,
        ],
        name="sc_gather_rows",
    )(table, idx3)


def _scatter_body(msgs_hbm, idx_hbm, base_hbm, out_hbm,
                  idx_v, buf, acc_sh, l0, l1, s0, s1):
    c = lax.axis_index("c")
    s = lax.axis_index("s")
    col0 = pl.multiple_of(c * _HALF, _HALF)
    n_rows = base_hbm.shape[0]
    nrow_chunks = n_rows // _SC
    my_rows = (nrow_chunks - s + _NSUB - 1) // _NSUB
    lsem = (l0, l1)
    ssem = (s0, s1)

    def init_body(i, carry):
        r0 = pl.multiple_of((s + i * _NSUB) * _SC, 8)
        pltpu.sync_copy(base_hbm.at[pl.ds(r0, _SC), pl.ds(col0, _HALF)],
                        buf.at[0])
        pltpu.sync_copy(buf.at[0], acc_sh.at[pl.ds(r0, _SC)])
        return carry

    lax.fori_loop(0, my_rows, init_body, 0)
    pltpu.sync_copy(idx_hbm.at[s], idx_v)
    plsc.subcore_barrier()

    e0 = s * (_E // _NSUB)

    def fire_load(ch, b):
        base = pl.multiple_of(e0 + ch * _SC, 8)
        pltpu.async_copy(msgs_hbm.at[pl.ds(base, _SC), pl.ds(col0, _HALF)],
                         buf.at[b], lsem[b])

    def drain_load(b):
        pltpu.make_async_copy(
            msgs_hbm.at[pl.ds(0, _SC), pl.ds(col0, _HALF)], buf.at[b],
            lsem[b]).wait()

    def fire_scat(ch, b):
        pltpu.async_copy(buf.at[b], acc_sh.at[idx_v.at[ch]], ssem[b],
                         add=True)

    def drain_scat(b):
        pltpu.make_async_copy(buf.at[b], acc_sh.at[idx_v.at[0]],
                              ssem[b]).wait()

    fire_load(0, 0)
    fire_load(1, 1)

    @pl.loop(2, _SN - 1, step=2)
    def _pipe(ch0):
        for b in (0, 1):
            ch = ch0 + b
            drain_load(b)
            fire_scat(ch - 2, b)
            drain_scat(b)
            fire_load(ch, b)

    drain_load(0)
    fire_scat(_SN - 3, 0)
    drain_scat(0)
    fire_load(_SN - 1, 0)
    drain_load(1)
    fire_scat(_SN - 2, 1)
    drain_load(0)
    fire_scat(_SN - 1, 0)
    drain_scat(1)
    drain_scat(0)
    plsc.subcore_barrier()

    def out_body(i, carry):
        r0 = pl.multiple_of((s + i * _NSUB) * _SC, 8)
        pltpu.sync_copy(acc_sh.at[pl.ds(r0, _SC)], buf.at[0])
        pltpu.sync_copy(buf.at[0],
                        out_hbm.at[pl.ds(r0, _SC), pl.ds(col0, _HALF)])
        return carry

    lax.fori_loop(0, my_rows, out_body, 0)


def _sc_scatter_add(msgs, idx3, base):
    n = base.shape[0]
    return pl.kernel(
        _scatter_body,
        out_type=jax.ShapeDtypeStruct((n, _D), jnp.float32),
        mesh=_mesh(),
        scratch_types=[
            pltpu.VMEM((_SN, _SC), jnp.int32),
            pltpu.VMEM((2, _SC, _HALF), jnp.float32),
            pltpu.VMEM_SHARED((n, _HALF), jnp.float32),
            pltpu.SemaphoreType.DMA,
            pltpu.SemaphoreType.DMA,
            pltpu.SemaphoreType.DMA,
            pltpu.SemaphoreType.DMA,
        ],
        name="sc_scatter_add",
    )(msgs, idx3, base)


def _layernorm(x, g, b, eps=1e-5):
    mu = jnp.mean(x, axis=-1, keepdims=True)
    var = jnp.mean((x - mu) ** 2, axis=-1, keepdims=True)
    return (x - mu) / jnp.sqrt(var + eps) * g + b


def _mlp_block(g_ref, f_ref, w1h_ref, w1f_ref, b1_ref, w2_ref, b2_ref,
               gam_ref, bet_ref, o_ref, *, fuse_ln):
    x = g_ref[...]
    if fuse_ln:
        x = _layernorm(x, gam_ref[...], bet_ref[...])
    a = (jnp.dot(x, w1h_ref[...], preferred_element_type=jnp.float32)
         + jnp.dot(f_ref[...], w1f_ref[...], preferred_element_type=jnp.float32)
         + b1_ref[...])
    h = a * jax.nn.sigmoid(a)
    o_ref[...] = (jnp.dot(h, w2_ref[...], preferred_element_type=jnp.float32)
                  + b2_ref[...])


def _tc_mlp(g, feat, w1h, w1f, b1, w2, b2, gam, bet, fuse_ln, block_e=1280):
    grid = _E // block_e
    row2 = lambda i: (0, 0)
    return pl.pallas_call(
        functools.partial(_mlp_block, fuse_ln=fuse_ln),
        grid=(grid,),
        in_specs=[
            pl.BlockSpec((block_e, _D), lambda i: (i, 0)),
            pl.BlockSpec((block_e, _EF), lambda i: (i, 0)),
            pl.BlockSpec((_D, _D), row2),
            pl.BlockSpec((_EF, _D), row2),
            pl.BlockSpec((1, _D), row2),
            pl.BlockSpec((_D, _D), row2),
            pl.BlockSpec((1, _D), row2),
            pl.BlockSpec((1, _D), row2),
            pl.BlockSpec((1, _D), row2),
        ],
        out_specs=pl.BlockSpec((block_e, _D), lambda i: (i, 0)),
        out_shape=jax.ShapeDtypeStruct((_E, _D), jnp.float32),
        name="tc_edge_mlp",
    )(g, feat, w1h, w1f, b1, w2, b2, gam, bet)


def _ln_block(x_ref, gam_ref, bet_ref, o_ref):
    o_ref[...] = _layernorm(x_ref[...], gam_ref[...], bet_ref[...])


def _tc_layernorm(x, gam, bet, block_n=2000):
    n = x.shape[0]
    grid = n // block_n
    return pl.pallas_call(
        _ln_block,
        grid=(grid,),
        in_specs=[
            pl.BlockSpec((block_n, _D), lambda i: (i, 0)),
            pl.BlockSpec((1, _D), lambda i: (0, 0)),
            pl.BlockSpec((1, _D), lambda i: (0, 0)),
        ],
        out_specs=pl.BlockSpec((block_n, _D), lambda i: (i, 0)),
        out_shape=jax.ShapeDtypeStruct((n, _D), jnp.float32),
        name="tc_layernorm",
    )(x, gam, bet)


def kernel(h_species, h_reactions, rs_feat, sr_feat,
           W1_rs, b1_rs, W2_rs, b2_rs, g_s, be_s,
           W1_sr, b1_sr, W2_sr, b2_sr, g_r, be_r,
           rs_index, sr_index):
    w1h_rs, w1f_rs = W1_rs[:_D], W1_rs[_D:]
    w1h_sr, w1f_sr = W1_sr[:_D], W1_sr[_D:]
    b1_rs2, b2_rs2 = b1_rs[None, :], b2_rs[None, :]
    b1_sr2, b2_sr2 = b1_sr[None, :], b2_sr[None, :]
    g_s2, be_s2 = g_s[None, :], be_s[None, :]
    g_r2, be_r2 = g_r[None, :], be_r[None, :]

    rs_src = rs_index[0].reshape(_NW, _GN, _GC)
    sr_src = sr_index[0].reshape(_NW, _GN, _GC)
    rs_dst = rs_index[1].reshape(_NSUB, _SN, _SC)
    sr_dst = sr_index[1].reshape(_NSUB, _SN, _SC)

    g1 = _sc_gather(h_reactions, rs_src)
    msgs1 = _tc_mlp(g1, rs_feat, w1h_rs, w1f_rs, b1_rs2, W2_rs, b2_rs2,
                    g_s2, be_s2, fuse_ln=False)
    s_raw = _sc_scatter_add(msgs1, rs_dst, h_species)

    g2 = _sc_gather(s_raw, sr_src)
    msgs2 = _tc_mlp(g2, sr_feat, w1h_sr, w1f_sr, b1_sr2, W2_sr, b2_sr2,
                    g_s2, be_s2, fuse_ln=True)
    r_raw = _sc_scatter_add(msgs2, sr_dst, h_reactions)

    h_species_out = _tc_layernorm(s_raw, g_s2, be_s2)
    h_reactions_out = _tc_layernorm(r_raw, g_r2, be_r2)
    return (h_species_out, h_reactions_out)
